# Initial kernel scaffold; baseline (speedup 1.0000x reference)
import jax
import jax.numpy as jnp
from jax.experimental import pallas as pl


def kernel(input, gnn_x, edge_index, conv1_W, conv1_b, conv2_W, conv2_b, fc_W, fc_b, W_ii, W_gi, W_hi, b_i, W_if, W_gf, W_hf, b_f, W_ig, W_gg, W_hg, b_g, W_io, W_go, W_ho, b_o):
    raise NotImplementedError("write your pallas kernel here")


# speedup vs baseline: 29.6247x; 29.6247x over previous
import functools

import jax
import jax.numpy as jnp
from jax import lax
from jax.experimental import pallas as pl
from jax.experimental.pallas import tpu as pltpu
from jax.experimental.pallas import tpu_sc as plsc

N = 2048
GH = 32
H = 512
T = 8
E = 65536
NSUB = 16
NCHUNK = 32
CHUNK = 128
ROWS = N // NSUB

_PREC = lax.Precision.HIGHEST


def _sc_deg_body(dst_hbm, ones_hbm, zeros_hbm, deg_hbm, idx_v, ones_v, acc_sh):
    c = lax.axis_index("c")
    s = lax.axis_index("s")
    pltpu.sync_copy(ones_hbm, ones_v)

    @pl.loop(0, T // 2)
    def _step(k):
        t = 2 * k + c
        pltpu.sync_copy(zeros_hbm.at[pl.ds(s * ROWS, ROWS)],
                        acc_sh.at[pl.ds(s * ROWS, ROWS)])
        pltpu.sync_copy(dst_hbm.at[t, s], idx_v)
        plsc.subcore_barrier()

        @pl.loop(0, NCHUNK)
        def _chunk(j):
            pltpu.sync_copy(ones_v, acc_sh.at[idx_v.at[j]], add=True)

        plsc.subcore_barrier()
        pltpu.sync_copy(acc_sh.at[pl.ds(s * ROWS, ROWS)],
                        deg_hbm.at[t, pl.ds(s * ROWS, ROWS)])
        plsc.subcore_barrier()


def _sc_msg_body(y_hbm, src_hbm, dst_hbm, zeros_hbm, out_hbm,
                 src_v, dst_v, rows_v, y_sh, acc_sh):
    c = lax.axis_index("c")
    s = lax.axis_index("s")

    @pl.loop(0, T // 2)
    def _step(k):
        t = 2 * k + c
        pltpu.sync_copy(zeros_hbm.at[pl.ds(s * ROWS, ROWS)],
                        acc_sh.at[pl.ds(s * ROWS, ROWS)])
        pltpu.sync_copy(y_hbm.at[t, pl.ds(s * ROWS, ROWS)],
                        y_sh.at[pl.ds(s * ROWS, ROWS)])
        pltpu.sync_copy(src_hbm.at[t, s], src_v)
        pltpu.sync_copy(dst_hbm.at[t, s], dst_v)
        plsc.subcore_barrier()

        @pl.loop(0, NCHUNK)
        def _chunk(j):
            pltpu.sync_copy(y_sh.at[src_v.at[j]], rows_v)
            pltpu.sync_copy(rows_v, acc_sh.at[dst_v.at[j]], add=True)

        plsc.subcore_barrier()
        pltpu.sync_copy(acc_sh.at[pl.ds(s * ROWS, ROWS)],
                        out_hbm.at[t, pl.ds(s * ROWS, ROWS)])
        plsc.subcore_barrier()


_sc_cache = {}

_SC_PARAMS = pltpu.CompilerParams(use_tc_tiling_on_sc=False)


def _sc_mesh():
    return plsc.VectorSubcoreMesh(core_axis_name="c", subcore_axis_name="s",
                                  num_cores=2, num_subcores=NSUB)


def _sc_deg(dst_r, ones16, zeros16):
    if "deg" not in _sc_cache:
        _sc_cache["deg"] = pl.kernel(
            _sc_deg_body,
            out_type=jax.ShapeDtypeStruct((T, N, 16), jnp.float32),
            mesh=_sc_mesh(),
            compiler_params=_SC_PARAMS,
            scratch_types=[
                pltpu.VMEM((NCHUNK, CHUNK), jnp.int32),
                pltpu.VMEM((CHUNK, 16), jnp.float32),
                pltpu.VMEM_SHARED((N, 16), jnp.float32),
            ],
        )
    return _sc_cache["deg"](dst_r, ones16, zeros16)


def _sc_msg(y, src_r, dst_r, zeros32):
    if "msg" not in _sc_cache:
        _sc_cache["msg"] = pl.kernel(
            _sc_msg_body,
            out_type=jax.ShapeDtypeStruct((T, N, GH), jnp.float32),
            mesh=_sc_mesh(),
            compiler_params=_SC_PARAMS,
            scratch_types=[
                pltpu.VMEM((NCHUNK, CHUNK), jnp.int32),
                pltpu.VMEM((NCHUNK, CHUNK), jnp.int32),
                pltpu.VMEM((CHUNK, GH), jnp.float32),
                pltpu.VMEM_SHARED((N, GH), jnp.float32),
                pltpu.VMEM_SHARED((N, GH), jnp.float32),
            ],
        )
    return _sc_cache["msg"](y, src_r, dst_r, zeros32)


def _mm_body(x_ref, w_ref, o_ref):
    o_ref[...] = jnp.dot(x_ref[...], w_ref[...],
                         preferred_element_type=jnp.float32, precision=_PREC)


def _big_matmul(x2d, w):
    bm = 1024
    return pl.pallas_call(
        _mm_body,
        grid=(x2d.shape[0] // bm,),
        in_specs=[
            pl.BlockSpec((bm, x2d.shape[1]), lambda i: (i, 0)),
            pl.BlockSpec((x2d.shape[1], w.shape[1]), lambda i: (0, 0)),
        ],
        out_specs=pl.BlockSpec((bm, w.shape[1]), lambda i: (i, 0)),
        out_shape=jax.ShapeDtypeStruct((x2d.shape[0], w.shape[1]), jnp.float32),
    )(x2d, w)


def _scale_body(deg_ref, xw_ref, y_ref):
    dinv = lax.rsqrt(deg_ref[0, :, 0:1] + 1.0)
    y_ref[0] = dinv * xw_ref[0]


def _scale_y(deg16, xw3):
    return pl.pallas_call(
        _scale_body,
        grid=(T,),
        in_specs=[
            pl.BlockSpec((1, N, 16), lambda t: (t, 0, 0)),
            pl.BlockSpec((1, N, GH), lambda t: (t, 0, 0)),
        ],
        out_specs=pl.BlockSpec((1, N, GH), lambda t: (t, 0, 0)),
        out_shape=jax.ShapeDtypeStruct((T, N, GH), jnp.float32),
    )(deg16, xw3)


def _mid_body(deg_ref, s1_ref, y1_ref, w2_ref, b1_ref, y2_ref):
    dinv = lax.rsqrt(deg_ref[0, :, 0:1] + 1.0)
    h1 = jax.nn.relu(dinv * (s1_ref[0] + y1_ref[0]) + b1_ref[...])
    xw2 = jnp.dot(h1, w2_ref[...],
                  preferred_element_type=jnp.float32, precision=_PREC)
    y2_ref[0] = dinv * xw2


def _mid(deg16, s1, y1, w2, b1):
    return pl.pallas_call(
        _mid_body,
        grid=(T,),
        in_specs=[
            pl.BlockSpec((1, N, 16), lambda t: (t, 0, 0)),
            pl.BlockSpec((1, N, GH), lambda t: (t, 0, 0)),
            pl.BlockSpec((1, N, GH), lambda t: (t, 0, 0)),
            pl.BlockSpec((GH, GH), lambda t: (0, 0)),
            pl.BlockSpec((1, GH), lambda t: (0, 0)),
        ],
        out_specs=pl.BlockSpec((1, N, GH), lambda t: (t, 0, 0)),
        out_shape=jax.ShapeDtypeStruct((T, N, GH), jnp.float32),
    )(deg16, s1, y1, w2, b1.reshape(1, GH))


def _out2_body(deg_ref, s2_ref, y2_ref, b2_ref, o_ref):
    dinv = lax.rsqrt(deg_ref[0, :, 0:1] + 1.0)
    o_ref[0] = dinv * (s2_ref[0] + y2_ref[0]) + b2_ref[...]


def _out2(deg16, s2, y2, b2):
    return pl.pallas_call(
        _out2_body,
        grid=(T,),
        in_specs=[
            pl.BlockSpec((1, N, 16), lambda t: (t, 0, 0)),
            pl.BlockSpec((1, N, GH), lambda t: (t, 0, 0)),
            pl.BlockSpec((1, N, GH), lambda t: (t, 0, 0)),
            pl.BlockSpec((1, GH), lambda t: (0, 0)),
        ],
        out_specs=pl.BlockSpec((1, N, GH), lambda t: (t, 0, 0)),
        out_shape=jax.ShapeDtypeStruct((T, N, GH), jnp.float32),
    )(deg16, s2, y2, b2.reshape(1, GH))


def _lstm_body(out2f_ref, fcw_ref, fcb_ref, inp_ref, wi_ref, wg_ref, wh_ref,
               ball_ref, outs_ref, c_ref):
    gf = jax.nn.relu(
        jnp.dot(out2f_ref[...], fcw_ref[...],
                preferred_element_type=jnp.float32, precision=_PREC)
        + fcb_ref[...])
    pre = (jnp.dot(inp_ref[...], wi_ref[...],
                   preferred_element_type=jnp.float32, precision=_PREC)
           + jnp.dot(gf, wg_ref[...],
                     preferred_element_type=jnp.float32, precision=_PREC)
           + ball_ref[...])
    h = jnp.zeros((1, H), jnp.float32)
    c = jnp.zeros((1, H), jnp.float32)
    for t in range(T):
        z = pre[t:t + 1, :] + jnp.dot(h, wh_ref[...],
                                      preferred_element_type=jnp.float32,
                                      precision=_PREC)
        i_t = jax.nn.sigmoid(z[:, 0:H])
        f_t = jax.nn.sigmoid(z[:, H:2 * H])
        g_t = jnp.tanh(z[:, 2 * H:3 * H])
        o_t = jax.nn.sigmoid(z[:, 3 * H:4 * H])
        c = f_t * c + i_t * g_t
        h = o_t * jnp.tanh(c)
        outs_ref[t:t + 1, :] = h
    c_ref[...] = c


def _lstm(out2f, fc_W, fc_b, inp2d, wi_all, wg_all, wh_all, b_all):
    return pl.pallas_call(
        _lstm_body,
        out_shape=(
            jax.ShapeDtypeStruct((T, H), jnp.float32),
            jax.ShapeDtypeStruct((1, H), jnp.float32),
        ),
    )(out2f, fc_W, fc_b.reshape(1, -1), inp2d, wi_all, wg_all, wh_all,
      b_all.reshape(1, -1))


def kernel(input, gnn_x, edge_index, conv1_W, conv1_b, conv2_W, conv2_b,
           fc_W, fc_b,
           W_ii, W_gi, W_hi, b_i,
           W_if, W_gf, W_hf, b_f,
           W_ig, W_gg, W_hg, b_g,
           W_io, W_go, W_ho, b_o):
    src_r = edge_index[:, 0].reshape(T, NSUB, NCHUNK, CHUNK)
    dst_r = edge_index[:, 1].reshape(T, NSUB, NCHUNK, CHUNK)
    ones16 = jnp.ones((CHUNK, 16), jnp.float32)
    zeros16 = jnp.zeros((N, 16), jnp.float32)
    zeros32 = jnp.zeros((N, GH), jnp.float32)

    deg16 = _sc_deg(dst_r, ones16, zeros16)
    xw3 = _big_matmul(gnn_x.reshape(T * N, N), conv1_W).reshape(T, N, GH)

    y1 = _scale_y(deg16, xw3)
    s1 = _sc_msg(y1, src_r, dst_r, zeros32)
    y2 = _mid(deg16, s1, y1, conv2_W, conv1_b)
    s2 = _sc_msg(y2, src_r, dst_r, zeros32)
    out2 = _out2(deg16, s2, y2, conv2_b)

    wi_all = jnp.concatenate([W_ii, W_if, W_ig, W_io], axis=1)
    wg_all = jnp.concatenate([W_gi, W_gf, W_gg, W_go], axis=1)
    wh_all = jnp.concatenate([W_hi, W_hf, W_hg, W_ho], axis=1)
    b_all = jnp.concatenate([b_i, b_f, b_g, b_o], axis=0)

    outs, c_fin = _lstm(out2.reshape(T, N * GH), fc_W, fc_b,
                        input.reshape(T, -1), wi_all, wg_all, wh_all, b_all)
    outputs = outs.reshape(T, 1, H)
    return (outputs, outs[T - 1:T], c_fin)


# speedup vs baseline: 33.1183x; 1.1179x over previous
import functools

import jax
import jax.numpy as jnp
from jax import lax
from jax.experimental import pallas as pl
from jax.experimental.pallas import tpu as pltpu
from jax.experimental.pallas import tpu_sc as plsc

N = 2048
GH = 32
H = 512
T = 8
E = 65536
NSUB = 16
NCHUNK = 32
CHUNK = 128
ROWS = N // NSUB

_PREC = lax.Precision.HIGHEST


def _sc_deg_body(dst_hbm, ones_hbm, zeros_hbm, deg_hbm, idx_v, ones_v, acc_sh):
    c = lax.axis_index("c")
    s = lax.axis_index("s")
    pltpu.sync_copy(ones_hbm, ones_v)

    @pl.loop(0, T // 2)
    def _step(k):
        t = 2 * k + c
        pltpu.sync_copy(zeros_hbm.at[pl.ds(s * ROWS, ROWS)],
                        acc_sh.at[pl.ds(s * ROWS, ROWS)])
        pltpu.sync_copy(dst_hbm.at[t, s], idx_v)
        plsc.subcore_barrier()

        @pl.loop(0, NCHUNK)
        def _chunk(j):
            pltpu.sync_copy(ones_v, acc_sh.at[idx_v.at[j]], add=True)

        plsc.subcore_barrier()
        pltpu.sync_copy(acc_sh.at[pl.ds(s * ROWS, ROWS)],
                        deg_hbm.at[t, pl.ds(s * ROWS, ROWS)])
        plsc.subcore_barrier()


_NSLOT = 8
_DEPTH = 4


def _sc_msg_body(y_hbm, src_hbm, dst_hbm, zeros_hbm, out_hbm,
                 src_v, dst_v, rows_v, y_sh, acc_sh, gsem, ssem):
    c = lax.axis_index("c")
    s = lax.axis_index("s")

    def wait_one(sem):
        pltpu.make_async_copy(y_hbm.at[0, pl.ds(0, CHUNK)],
                              rows_v.at[0], sem).wait()

    @pl.loop(0, T // 2)
    def _step(k):
        t = 2 * k + c
        pltpu.sync_copy(zeros_hbm.at[pl.ds(s * ROWS, ROWS)],
                        acc_sh.at[pl.ds(s * ROWS, ROWS)])
        pltpu.sync_copy(y_hbm.at[t, pl.ds(s * ROWS, ROWS)],
                        y_sh.at[pl.ds(s * ROWS, ROWS)])
        pltpu.sync_copy(src_hbm.at[t, s], src_v)
        pltpu.sync_copy(dst_hbm.at[t, s], dst_v)
        plsc.subcore_barrier()

        for b in range(_DEPTH):
            pltpu.async_copy(y_sh.at[src_v.at[b]], rows_v.at[b], gsem)

        @pl.loop(0, NCHUNK // _NSLOT)
        def _grp(g):
            for jj in range(_NSLOT):
                j = g * _NSLOT + jj
                b = jj
                nb = (jj + _DEPTH) % _NSLOT
                wait_one(gsem)
                pltpu.async_copy(rows_v.at[b], acc_sh.at[dst_v.at[j]],
                                 ssem, add=True)

                @pl.when(j >= _DEPTH)
                def _():
                    wait_one(ssem)

                @pl.when(j + _DEPTH < NCHUNK)
                def _():
                    pltpu.async_copy(y_sh.at[src_v.at[j + _DEPTH]],
                                     rows_v.at[nb], gsem)

        for _ in range(_DEPTH):
            wait_one(ssem)

        plsc.subcore_barrier()
        pltpu.sync_copy(acc_sh.at[pl.ds(s * ROWS, ROWS)],
                        out_hbm.at[t, pl.ds(s * ROWS, ROWS)])
        plsc.subcore_barrier()


_sc_cache = {}

_SC_PARAMS = pltpu.CompilerParams(use_tc_tiling_on_sc=False)


def _sc_mesh():
    return plsc.VectorSubcoreMesh(core_axis_name="c", subcore_axis_name="s",
                                  num_cores=2, num_subcores=NSUB)


def _sc_deg(dst_r, ones16, zeros16):
    if "deg" not in _sc_cache:
        _sc_cache["deg"] = pl.kernel(
            _sc_deg_body,
            out_type=jax.ShapeDtypeStruct((T, N, 16), jnp.float32),
            mesh=_sc_mesh(),
            compiler_params=_SC_PARAMS,
            scratch_types=[
                pltpu.VMEM((NCHUNK, CHUNK), jnp.int32),
                pltpu.VMEM((CHUNK, 16), jnp.float32),
                pltpu.VMEM_SHARED((N, 16), jnp.float32),
            ],
        )
    return _sc_cache["deg"](dst_r, ones16, zeros16)


def _sc_msg(y, src_r, dst_r, zeros32):
    if "msg" not in _sc_cache:
        _sc_cache["msg"] = pl.kernel(
            _sc_msg_body,
            out_type=jax.ShapeDtypeStruct((T, N, GH), jnp.float32),
            mesh=_sc_mesh(),
            compiler_params=_SC_PARAMS,
            scratch_types=[
                pltpu.VMEM((NCHUNK, CHUNK), jnp.int32),
                pltpu.VMEM((NCHUNK, CHUNK), jnp.int32),
                pltpu.VMEM((_NSLOT, CHUNK, GH), jnp.float32),
                pltpu.VMEM_SHARED((N, GH), jnp.float32),
                pltpu.VMEM_SHARED((N, GH), jnp.float32),
                pltpu.SemaphoreType.DMA,
                pltpu.SemaphoreType.DMA,
            ],
        )
    return _sc_cache["msg"](y, src_r, dst_r, zeros32)


def _mm_body(x_ref, w_ref, deg_ref, y_ref):
    dinv = lax.rsqrt(deg_ref[0, :, 0:1] + 1.0)
    y_ref[0] = dinv * jnp.dot(x_ref[0], w_ref[...],
                              preferred_element_type=jnp.float32,
                              precision=_PREC)


def _big_matmul(gnn_x, w, deg16):
    bm = 1024
    return pl.pallas_call(
        _mm_body,
        grid=(T, N // bm),
        in_specs=[
            pl.BlockSpec((1, bm, N), lambda t, i: (t, i, 0)),
            pl.BlockSpec((N, GH), lambda t, i: (0, 0)),
            pl.BlockSpec((1, bm, 16), lambda t, i: (t, i, 0)),
        ],
        out_specs=pl.BlockSpec((1, bm, GH), lambda t, i: (t, i, 0)),
        out_shape=jax.ShapeDtypeStruct((T, N, GH), jnp.float32),
    )(gnn_x, w, deg16)


def _mid_body(deg_ref, s1_ref, y1_ref, w2_ref, b1_ref, y2_ref):
    dinv = lax.rsqrt(deg_ref[0, :, 0:1] + 1.0)
    h1 = jax.nn.relu(dinv * (s1_ref[0] + y1_ref[0]) + b1_ref[...])
    xw2 = jnp.dot(h1, w2_ref[...],
                  preferred_element_type=jnp.float32, precision=_PREC)
    y2_ref[0] = dinv * xw2


def _mid(deg16, s1, y1, w2, b1):
    return pl.pallas_call(
        _mid_body,
        grid=(T,),
        in_specs=[
            pl.BlockSpec((1, N, 16), lambda t: (t, 0, 0)),
            pl.BlockSpec((1, N, GH), lambda t: (t, 0, 0)),
            pl.BlockSpec((1, N, GH), lambda t: (t, 0, 0)),
            pl.BlockSpec((GH, GH), lambda t: (0, 0)),
            pl.BlockSpec((1, GH), lambda t: (0, 0)),
        ],
        out_specs=pl.BlockSpec((1, N, GH), lambda t: (t, 0, 0)),
        out_shape=jax.ShapeDtypeStruct((T, N, GH), jnp.float32),
    )(deg16, s1, y1, w2, b1.reshape(1, GH))


def _out2_body(deg_ref, s2_ref, y2_ref, b2_ref, o_ref):
    dinv = lax.rsqrt(deg_ref[0, :, 0:1] + 1.0)
    o_ref[0] = dinv * (s2_ref[0] + y2_ref[0]) + b2_ref[...]


def _out2(deg16, s2, y2, b2):
    return pl.pallas_call(
        _out2_body,
        grid=(T,),
        in_specs=[
            pl.BlockSpec((1, N, 16), lambda t: (t, 0, 0)),
            pl.BlockSpec((1, N, GH), lambda t: (t, 0, 0)),
            pl.BlockSpec((1, N, GH), lambda t: (t, 0, 0)),
            pl.BlockSpec((1, GH), lambda t: (0, 0)),
        ],
        out_specs=pl.BlockSpec((1, N, GH), lambda t: (t, 0, 0)),
        out_shape=jax.ShapeDtypeStruct((T, N, GH), jnp.float32),
    )(deg16, s2, y2, b2.reshape(1, GH))


def _lstm_body(out2f_ref, fcw_ref, fcb_ref, inp_ref, wi_ref, wg_ref, wh_ref,
               ball_ref, outs_ref, c_ref):
    gf = jax.nn.relu(
        jnp.dot(out2f_ref[...], fcw_ref[...],
                preferred_element_type=jnp.float32, precision=_PREC)
        + fcb_ref[...])
    pre = (jnp.dot(inp_ref[...], wi_ref[...],
                   preferred_element_type=jnp.float32, precision=_PREC)
           + jnp.dot(gf, wg_ref[...],
                     preferred_element_type=jnp.float32, precision=_PREC)
           + ball_ref[...])
    h = jnp.zeros((1, H), jnp.float32)
    c = jnp.zeros((1, H), jnp.float32)
    for t in range(T):
        z = pre[t:t + 1, :] + jnp.dot(h, wh_ref[...],
                                      preferred_element_type=jnp.float32,
                                      precision=_PREC)
        i_t = jax.nn.sigmoid(z[:, 0:H])
        f_t = jax.nn.sigmoid(z[:, H:2 * H])
        g_t = jnp.tanh(z[:, 2 * H:3 * H])
        o_t = jax.nn.sigmoid(z[:, 3 * H:4 * H])
        c = f_t * c + i_t * g_t
        h = o_t * jnp.tanh(c)
        outs_ref[t:t + 1, :] = h
    c_ref[...] = c


def _lstm(out2f, fc_W, fc_b, inp2d, wi_all, wg_all, wh_all, b_all):
    return pl.pallas_call(
        _lstm_body,
        out_shape=(
            jax.ShapeDtypeStruct((T, H), jnp.float32),
            jax.ShapeDtypeStruct((1, H), jnp.float32),
        ),
    )(out2f, fc_W, fc_b.reshape(1, -1), inp2d, wi_all, wg_all, wh_all,
      b_all.reshape(1, -1))


def kernel(input, gnn_x, edge_index, conv1_W, conv1_b, conv2_W, conv2_b,
           fc_W, fc_b,
           W_ii, W_gi, W_hi, b_i,
           W_if, W_gf, W_hf, b_f,
           W_ig, W_gg, W_hg, b_g,
           W_io, W_go, W_ho, b_o):
    src_r = edge_index[:, 0].reshape(T, NSUB, NCHUNK, CHUNK)
    dst_r = edge_index[:, 1].reshape(T, NSUB, NCHUNK, CHUNK)
    ones16 = jnp.ones((CHUNK, 16), jnp.float32)
    zeros16 = jnp.zeros((N, 16), jnp.float32)
    zeros32 = jnp.zeros((N, GH), jnp.float32)

    deg16 = _sc_deg(dst_r, ones16, zeros16)
    y1 = _big_matmul(gnn_x, conv1_W, deg16)
    s1 = _sc_msg(y1, src_r, dst_r, zeros32)
    y2 = _mid(deg16, s1, y1, conv2_W, conv1_b)
    s2 = _sc_msg(y2, src_r, dst_r, zeros32)
    out2 = _out2(deg16, s2, y2, conv2_b)

    wi_all = jnp.concatenate([W_ii, W_if, W_ig, W_io], axis=1)
    wg_all = jnp.concatenate([W_gi, W_gf, W_gg, W_go], axis=1)
    wh_all = jnp.concatenate([W_hi, W_hf, W_hg, W_ho], axis=1)
    b_all = jnp.concatenate([b_i, b_f, b_g, b_o], axis=0)

    outs, c_fin = _lstm(out2.reshape(T, N * GH), fc_W, fc_b,
                        input.reshape(T, -1), wi_all, wg_all, wh_all, b_all)
    outputs = outs.reshape(T, 1, H)
    return (outputs, outs[T - 1:T], c_fin)


# speedup vs baseline: 39.9446x; 1.2061x over previous
import functools

import jax
import jax.numpy as jnp
from jax import lax
from jax.experimental import pallas as pl
from jax.experimental.pallas import tpu as pltpu
from jax.experimental.pallas import tpu_sc as plsc

N = 2048
GH = 32
H = 512
T = 8
E = 65536
NSUB = 16
NCHUNK = 32
CHUNK = 128
ROWS = N // NSUB

_PREC = lax.Precision.DEFAULT


def _sc_deg_body(dst_hbm, ones_hbm, zeros_hbm, deg_hbm, idx_v, ones_v, acc_sh):
    c = lax.axis_index("c")
    s = lax.axis_index("s")
    pltpu.sync_copy(ones_hbm, ones_v)

    @pl.loop(0, T // 2)
    def _step(k):
        t = 2 * k + c
        pltpu.sync_copy(zeros_hbm.at[pl.ds(s * ROWS, ROWS)],
                        acc_sh.at[pl.ds(s * ROWS, ROWS)])
        pltpu.sync_copy(dst_hbm.at[t, s], idx_v)
        plsc.subcore_barrier()

        @pl.loop(0, NCHUNK)
        def _chunk(j):
            pltpu.sync_copy(ones_v, acc_sh.at[idx_v.at[j]], add=True)

        plsc.subcore_barrier()
        pltpu.sync_copy(acc_sh.at[pl.ds(s * ROWS, ROWS)],
                        deg_hbm.at[t, pl.ds(s * ROWS, ROWS)])
        plsc.subcore_barrier()


_NSLOT = 8
_DEPTH = 4


def _sc_msg_body(y_hbm, src_hbm, dst_hbm, zeros_hbm, out_hbm,
                 src_v, dst_v, rows_v, y_sh, acc_sh, gsem, ssem):
    c = lax.axis_index("c")
    s = lax.axis_index("s")

    def wait_one(sem):
        pltpu.make_async_copy(y_hbm.at[0, pl.ds(0, CHUNK)],
                              rows_v.at[0], sem).wait()

    @pl.loop(0, T // 2)
    def _step(k):
        t = 2 * k + c
        pltpu.sync_copy(zeros_hbm.at[pl.ds(s * ROWS, ROWS)],
                        acc_sh.at[pl.ds(s * ROWS, ROWS)])
        pltpu.sync_copy(y_hbm.at[t, pl.ds(s * ROWS, ROWS)],
                        y_sh.at[pl.ds(s * ROWS, ROWS)])
        pltpu.sync_copy(src_hbm.at[t, s], src_v)
        pltpu.sync_copy(dst_hbm.at[t, s], dst_v)
        plsc.subcore_barrier()

        for b in range(_DEPTH):
            pltpu.async_copy(y_sh.at[src_v.at[b]], rows_v.at[b], gsem)

        @pl.loop(0, NCHUNK // _NSLOT)
        def _grp(g):
            for jj in range(_NSLOT):
                j = g * _NSLOT + jj
                b = jj
                nb = (jj + _DEPTH) % _NSLOT
                wait_one(gsem)
                pltpu.async_copy(rows_v.at[b], acc_sh.at[dst_v.at[j]],
                                 ssem, add=True)

                @pl.when(j >= _DEPTH)
                def _():
                    wait_one(ssem)

                @pl.when(j + _DEPTH < NCHUNK)
                def _():
                    pltpu.async_copy(y_sh.at[src_v.at[j + _DEPTH]],
                                     rows_v.at[nb], gsem)

        for _ in range(_DEPTH):
            wait_one(ssem)

        plsc.subcore_barrier()
        pltpu.sync_copy(acc_sh.at[pl.ds(s * ROWS, ROWS)],
                        out_hbm.at[t, pl.ds(s * ROWS, ROWS)])
        plsc.subcore_barrier()


_sc_cache = {}

_SC_PARAMS = pltpu.CompilerParams(use_tc_tiling_on_sc=False)


def _sc_mesh():
    return plsc.VectorSubcoreMesh(core_axis_name="c", subcore_axis_name="s",
                                  num_cores=2, num_subcores=NSUB)


def _sc_deg(dst_r, ones16, zeros16):
    if "deg" not in _sc_cache:
        _sc_cache["deg"] = pl.kernel(
            _sc_deg_body,
            out_type=jax.ShapeDtypeStruct((T, N, 16), jnp.float32),
            mesh=_sc_mesh(),
            compiler_params=_SC_PARAMS,
            scratch_types=[
                pltpu.VMEM((NCHUNK, CHUNK), jnp.int32),
                pltpu.VMEM((CHUNK, 16), jnp.float32),
                pltpu.VMEM_SHARED((N, 16), jnp.float32),
            ],
        )
    return _sc_cache["deg"](dst_r, ones16, zeros16)


def _sc_msg(y, src_r, dst_r, zeros32):
    if "msg" not in _sc_cache:
        _sc_cache["msg"] = pl.kernel(
            _sc_msg_body,
            out_type=jax.ShapeDtypeStruct((T, N, GH), jnp.float32),
            mesh=_sc_mesh(),
            compiler_params=_SC_PARAMS,
            scratch_types=[
                pltpu.VMEM((NCHUNK, CHUNK), jnp.int32),
                pltpu.VMEM((NCHUNK, CHUNK), jnp.int32),
                pltpu.VMEM((_NSLOT, CHUNK, GH), jnp.float32),
                pltpu.VMEM_SHARED((N, GH), jnp.float32),
                pltpu.VMEM_SHARED((N, GH), jnp.float32),
                pltpu.SemaphoreType.DMA,
                pltpu.SemaphoreType.DMA,
            ],
        )
    return _sc_cache["msg"](y, src_r, dst_r, zeros32)


def _mm_body(x_ref, w_ref, deg_ref, y_ref):
    dinv = lax.rsqrt(deg_ref[0, :, 0:1] + 1.0)
    y_ref[0] = dinv * jnp.dot(x_ref[0], w_ref[...],
                              preferred_element_type=jnp.float32,
                              precision=_PREC)


def _big_matmul(gnn_x, w, deg16):
    bm = 256
    return pl.pallas_call(
        _mm_body,
        grid=(T, N // bm),
        in_specs=[
            pl.BlockSpec((1, bm, N), lambda t, i: (t, i, 0)),
            pl.BlockSpec((N, GH), lambda t, i: (0, 0)),
            pl.BlockSpec((1, bm, 16), lambda t, i: (t, i, 0)),
        ],
        out_specs=pl.BlockSpec((1, bm, GH), lambda t, i: (t, i, 0)),
        out_shape=jax.ShapeDtypeStruct((T, N, GH), jnp.float32),
    )(gnn_x, w, deg16)


def _mid_body(deg_ref, s1_ref, y1_ref, w2_ref, b1_ref, y2_ref):
    dinv = lax.rsqrt(deg_ref[0, :, 0:1] + 1.0)
    h1 = jax.nn.relu(dinv * (s1_ref[0] + y1_ref[0]) + b1_ref[...])
    xw2 = jnp.dot(h1, w2_ref[...],
                  preferred_element_type=jnp.float32, precision=_PREC)
    y2_ref[0] = dinv * xw2


def _mid(deg16, s1, y1, w2, b1):
    return pl.pallas_call(
        _mid_body,
        grid=(T,),
        in_specs=[
            pl.BlockSpec((1, N, 16), lambda t: (t, 0, 0)),
            pl.BlockSpec((1, N, GH), lambda t: (t, 0, 0)),
            pl.BlockSpec((1, N, GH), lambda t: (t, 0, 0)),
            pl.BlockSpec((GH, GH), lambda t: (0, 0)),
            pl.BlockSpec((1, GH), lambda t: (0, 0)),
        ],
        out_specs=pl.BlockSpec((1, N, GH), lambda t: (t, 0, 0)),
        out_shape=jax.ShapeDtypeStruct((T, N, GH), jnp.float32),
    )(deg16, s1, y1, w2, b1.reshape(1, GH))


def _out2_body(deg_ref, s2_ref, y2_ref, b2_ref, o_ref):
    dinv = lax.rsqrt(deg_ref[0, :, 0:1] + 1.0)
    o_ref[0] = dinv * (s2_ref[0] + y2_ref[0]) + b2_ref[...]


def _out2(deg16, s2, y2, b2):
    return pl.pallas_call(
        _out2_body,
        grid=(T,),
        in_specs=[
            pl.BlockSpec((1, N, 16), lambda t: (t, 0, 0)),
            pl.BlockSpec((1, N, GH), lambda t: (t, 0, 0)),
            pl.BlockSpec((1, N, GH), lambda t: (t, 0, 0)),
            pl.BlockSpec((1, GH), lambda t: (0, 0)),
        ],
        out_specs=pl.BlockSpec((1, N, GH), lambda t: (t, 0, 0)),
        out_shape=jax.ShapeDtypeStruct((T, N, GH), jnp.float32),
    )(deg16, s2, y2, b2.reshape(1, GH))


def _lstm_body(out2f_ref, fcw_ref, fcb_ref, inp_ref, wi_ref, wg_ref, wh_ref,
               ball_ref, outs_ref, c_ref):
    gf = jax.nn.relu(
        jnp.dot(out2f_ref[...], fcw_ref[...],
                preferred_element_type=jnp.float32, precision=_PREC)
        + fcb_ref[...])
    pre = (jnp.dot(inp_ref[...], wi_ref[...],
                   preferred_element_type=jnp.float32, precision=_PREC)
           + jnp.dot(gf, wg_ref[...],
                     preferred_element_type=jnp.float32, precision=_PREC)
           + ball_ref[...])
    h = jnp.zeros((1, H), jnp.float32)
    c = jnp.zeros((1, H), jnp.float32)
    for t in range(T):
        z = pre[t:t + 1, :] + jnp.dot(h, wh_ref[...],
                                      preferred_element_type=jnp.float32,
                                      precision=_PREC)
        i_t = jax.nn.sigmoid(z[:, 0:H])
        f_t = jax.nn.sigmoid(z[:, H:2 * H])
        g_t = jnp.tanh(z[:, 2 * H:3 * H])
        o_t = jax.nn.sigmoid(z[:, 3 * H:4 * H])
        c = f_t * c + i_t * g_t
        h = o_t * jnp.tanh(c)
        outs_ref[t:t + 1, :] = h
    c_ref[...] = c


def _lstm(out2f, fc_W, fc_b, inp2d, wi_all, wg_all, wh_all, b_all):
    return pl.pallas_call(
        _lstm_body,
        out_shape=(
            jax.ShapeDtypeStruct((T, H), jnp.float32),
            jax.ShapeDtypeStruct((1, H), jnp.float32),
        ),
    )(out2f, fc_W, fc_b.reshape(1, -1), inp2d, wi_all, wg_all, wh_all,
      b_all.reshape(1, -1))


def kernel(input, gnn_x, edge_index, conv1_W, conv1_b, conv2_W, conv2_b,
           fc_W, fc_b,
           W_ii, W_gi, W_hi, b_i,
           W_if, W_gf, W_hf, b_f,
           W_ig, W_gg, W_hg, b_g,
           W_io, W_go, W_ho, b_o):
    src_r = edge_index[:, 0].reshape(T, NSUB, NCHUNK, CHUNK)
    dst_r = edge_index[:, 1].reshape(T, NSUB, NCHUNK, CHUNK)
    ones16 = jnp.ones((CHUNK, 16), jnp.float32)
    zeros16 = jnp.zeros((N, 16), jnp.float32)
    zeros32 = jnp.zeros((N, GH), jnp.float32)

    deg16 = _sc_deg(dst_r, ones16, zeros16)
    y1 = _big_matmul(gnn_x, conv1_W, deg16)
    s1 = _sc_msg(y1, src_r, dst_r, zeros32)
    y2 = _mid(deg16, s1, y1, conv2_W, conv1_b)
    s2 = _sc_msg(y2, src_r, dst_r, zeros32)
    out2 = _out2(deg16, s2, y2, conv2_b)

    wi_all = jnp.concatenate([W_ii, W_if, W_ig, W_io], axis=1)
    wg_all = jnp.concatenate([W_gi, W_gf, W_gg, W_go], axis=1)
    wh_all = jnp.concatenate([W_hi, W_hf, W_hg, W_ho], axis=1)
    b_all = jnp.concatenate([b_i, b_f, b_g, b_o], axis=0)

    outs, c_fin = _lstm(out2.reshape(T, N * GH), fc_W, fc_b,
                        input.reshape(T, -1), wi_all, wg_all, wh_all, b_all)
    outputs = outs.reshape(T, 1, H)
    return (outputs, outs[T - 1:T], c_fin)
